# baseline (speedup 1.0000x reference)
import functools

import jax
import jax.numpy as jnp
from jax import lax
from jax.experimental import pallas as pl
from jax.experimental.pallas import tpu as pltpu
from jax.experimental.pallas import tpu_sc as plsc


def _prep_body(ids_ref, state_ref, species_ref, fid_ref, comb_ref):
    comb_ref[...] = species_ref[...][:, None, :] + state_ref[...][None, :, :]
    ids_t = jnp.transpose(ids_ref[...], (0, 2, 1))
    s_iota = lax.broadcasted_iota(jnp.int32, ids_t.shape, 2)
    fid = ids_t + 3 * s_iota
    fid_ref[...] = fid.reshape(fid_ref.shape)


def _make_sc_gather(n_rows, h, per_w, chunk, num_cores):
    nch = per_w // chunk
    mesh = plsc.VectorSubcoreMesh(core_axis_name="c", subcore_axis_name="s")

    @functools.partial(
        pl.kernel,
        mesh=mesh,
        out_type=jax.ShapeDtypeStruct((n_rows, h), jnp.float32),
        scratch_types=[
            pltpu.VMEM((per_w,), jnp.int32),
            pltpu.VMEM((chunk, h), jnp.float32),
            pltpu.VMEM((chunk, h), jnp.float32),
            pltpu.VMEM((chunk, h), jnp.float32),
            pltpu.SemaphoreType.DMA,
            pltpu.SemaphoreType.DMA,
            pltpu.SemaphoreType.DMA,
            pltpu.SemaphoreType.DMA,
            pltpu.SemaphoreType.DMA,
            pltpu.SemaphoreType.DMA,
        ],
    )
    def sc_gather(fid_hbm, comb_hbm, out_hbm, idx_v, buf0, buf1, buf2,
                  sg0, sg1, sg2, ss0, ss1, ss2):
        wid = lax.axis_index("s") * num_cores + lax.axis_index("c")
        base = wid * per_w
        pltpu.sync_copy(fid_hbm.at[pl.ds(base, per_w)], idx_v)

        bufs = (buf0, buf1, buf2)
        gsems = (sg0, sg1, sg2)
        ssems = (ss0, ss1, ss2)

        def start_gather(c):
            b = c % 3
            return pltpu.async_copy(
                comb_hbm.at[idx_v.at[pl.ds(c * chunk, chunk)]], bufs[b], gsems[b]
            )

        def start_scatter(c):
            b = c % 3
            return pltpu.async_copy(
                bufs[b], out_hbm.at[pl.ds(base + c * chunk, chunk)], ssems[b]
            )

        g = [None] * nch
        s = [None] * nch
        for c in range(nch):
            if c >= 3:
                s[c - 3].wait()
            g[c] = start_gather(c)
            if c >= 1:
                g[c - 1].wait()
                s[c - 1] = start_scatter(c - 1)
        g[nch - 1].wait()
        s[nch - 1] = start_scatter(nch - 1)
        for c in range(max(0, nch - 3), nch - 1):
            s[c].wait()
        s[nch - 1].wait()

    return sc_gather


def kernel(input_ids, state_table, species_table):
    B, S, T = input_ids.shape
    K, H = state_table.shape
    ids = input_ids.astype(jnp.int32)

    fid2d, comb3d = pl.pallas_call(
        _prep_body,
        out_shape=(
            jax.ShapeDtypeStruct((B * T, S), jnp.int32),
            jax.ShapeDtypeStruct((S, K, H), jnp.float32),
        ),
    )(ids, state_table, species_table)

    fid = fid2d.reshape(B * T * S)
    comb = comb3d.reshape(S * K, H)

    n_rows = B * T * S
    info = plsc.get_sparse_core_info()
    nw = info.num_cores * info.num_subcores
    per_w = n_rows // nw
    chunk = 160

    sc_gather = _make_sc_gather(n_rows, H, per_w, chunk, info.num_cores)
    out2d = sc_gather(fid, comb)
    return jnp.swapaxes(out2d.reshape(B, T, S, H), 1, 2)


# speedup vs baseline: 3.3931x; 3.3931x over previous
import jax
import jax.numpy as jnp
from jax.experimental import pallas as pl


def _tc_body(ids_ref, state_ref, species_ref, out_ref):
    ids = ids_ref[...]
    st = state_ref[...]
    sp = species_ref[...]
    ids_t = jnp.transpose(ids[0], (1, 0))
    idsx = ids_t[:, :, None]
    r0 = st[0][None, None, :]
    r1 = st[1][None, None, :]
    r2 = st[2][None, None, :]
    state_emb = jnp.where(idsx == 0, r0, jnp.where(idsx == 1, r1, r2))
    out_ref[...] = (state_emb + sp[None, :, :])[None]


def kernel(input_ids, state_table, species_table):
    B, S, T = input_ids.shape
    H = state_table.shape[1]
    ids = input_ids.astype(jnp.int32)
    Sb = 256
    s_blocks = S // Sb
    out_t = pl.pallas_call(
        _tc_body,
        grid=(B, s_blocks),
        in_specs=[
            pl.BlockSpec((1, Sb, T), lambda b, j: (b, j, 0)),
            pl.BlockSpec((3, H), lambda b, j: (0, 0)),
            pl.BlockSpec((Sb, H), lambda b, j: (j, 0)),
        ],
        out_specs=pl.BlockSpec((1, T, Sb, H), lambda b, j: (b, 0, j, 0)),
        out_shape=jax.ShapeDtypeStruct((B, T, S, H), jnp.float32),
    )(ids, state_table, species_table)
    return jnp.swapaxes(out_t, 1, 2)
